# baseline (speedup 1.0000x reference)
import functools

import jax
import jax.numpy as jnp
from jax import lax
from jax.experimental import pallas as pl
from jax.experimental.pallas import tpu as pltpu
from jax.experimental.pallas import tpu_sc as plsc

EMB = 64
BATCH = 16384


@jax.jit
def _presuf_embed(pre, suf, W_pre, W_suf):
    info = plsc.get_sparse_core_info()
    nw = info.num_cores * info.num_subcores
    bpw = BATCH // nw

    mesh = plsc.VectorSubcoreMesh(core_axis_name="c", subcore_axis_name="s")

    @functools.partial(
        pl.kernel,
        mesh=mesh,
        compiler_params=pltpu.CompilerParams(use_tc_tiling_on_sc=False),
        out_type=jax.ShapeDtypeStruct((BATCH, 2, EMB), jnp.float32),
        scratch_types=[
            pltpu.VMEM((bpw,), jnp.int32),
            pltpu.VMEM((bpw,), jnp.int32),
            pltpu.VMEM((bpw, EMB), jnp.float32),
            pltpu.VMEM((bpw, EMB), jnp.float32),
            pltpu.SemaphoreType.DMA,
            pltpu.SemaphoreType.DMA,
        ],
    )
    def k(pre_hbm, suf_hbm, wpre_hbm, wsuf_hbm, out_hbm,
          pidx_v, sidx_v, prow_v, srow_v, sem0, sem1):
        wid = lax.axis_index("s") * info.num_cores + lax.axis_index("c")
        base = wid * bpw
        pltpu.sync_copy(pre_hbm.at[pl.ds(base, bpw)], pidx_v)
        pltpu.sync_copy(suf_hbm.at[pl.ds(base, bpw)], sidx_v)
        cp0 = pltpu.async_copy(wpre_hbm.at[pidx_v], prow_v, sem0)
        cp1 = pltpu.async_copy(wsuf_hbm.at[sidx_v], srow_v, sem1)
        cp0.wait()
        cp1.wait()
        pltpu.sync_copy(prow_v, out_hbm.at[pl.ds(base, bpw), 0])
        pltpu.sync_copy(srow_v, out_hbm.at[pl.ds(base, bpw), 1])

    out = k(pre, suf, W_pre, W_suf)
    return out.reshape(BATCH, 2 * EMB)


def kernel(unused, pre, suf, W_pre, W_suf):
    return _presuf_embed(pre, suf, W_pre, W_suf)


# speedup vs baseline: 1.0373x; 1.0373x over previous
import functools

import jax
import jax.numpy as jnp
from jax import lax
from jax.experimental import pallas as pl
from jax.experimental.pallas import tpu as pltpu
from jax.experimental.pallas import tpu_sc as plsc

EMB = 64
BATCH = 16384


@jax.jit
def _presuf_embed(pre, suf, W_pre, W_suf):
    info = plsc.get_sparse_core_info()
    nw = info.num_cores * info.num_subcores
    bpw = BATCH // nw
    chunk = 128
    nchunks = bpw // chunk

    pre2 = W_pre.reshape(500000, 2 * EMB)
    suf2 = W_suf.reshape(500000, 2 * EMB)

    mesh = plsc.VectorSubcoreMesh(core_axis_name="c", subcore_axis_name="s")

    @functools.partial(
        pl.kernel,
        mesh=mesh,
        out_type=jax.ShapeDtypeStruct((BATCH, 2 * EMB), jnp.float32),
        scratch_types=[
            pltpu.VMEM((bpw,), jnp.int32),
            pltpu.VMEM((bpw,), jnp.int32),
            pltpu.VMEM((bpw,), jnp.int32),
            pltpu.VMEM((bpw,), jnp.int32),
            pltpu.VMEM((chunk, 2 * EMB), jnp.float32),
            pltpu.VMEM((chunk, 2 * EMB), jnp.float32),
            pltpu.VMEM((chunk, 2 * EMB), jnp.float32),
            pltpu.SemaphoreType.DMA,
            pltpu.SemaphoreType.DMA,
        ],
    )
    def k(pre_hbm, suf_hbm, wpre_hbm, wsuf_hbm, out_hbm,
          pidx_v, sidx_v, ptid_v, stid_v, gpre_v, gsuf_v, outb_v,
          sem0, sem1):
        wid = lax.axis_index("s") * info.num_cores + lax.axis_index("c")
        base = wid * bpw
        pltpu.sync_copy(pre_hbm.at[pl.ds(base, bpw)], pidx_v)
        pltpu.sync_copy(suf_hbm.at[pl.ds(base, bpw)], sidx_v)

        def shift(t):
            ptid_v[pl.ds(t * 16, 16)] = pidx_v[pl.ds(t * 16, 16)] >> 1
            stid_v[pl.ds(t * 16, 16)] = sidx_v[pl.ds(t * 16, 16)] >> 1

        pl.loop(0, bpw // 16)(shift)

        def do_chunk(c):
            cb = c * chunk
            cp0 = pltpu.async_copy(
                wpre_hbm.at[ptid_v.at[pl.ds(cb, chunk)]], gpre_v, sem0)
            cp1 = pltpu.async_copy(
                wsuf_hbm.at[stid_v.at[pl.ds(cb, chunk)]], gsuf_v, sem1)
            cp0.wait()
            cp1.wait()

            def row16(t):
                pv = (pidx_v[pl.ds(cb + t * 16, 16)] & 1) * EMB
                sv = (sidx_v[pl.ds(cb + t * 16, 16)] & 1) * EMB
                for i in range(16):
                    po = pv[i]
                    so = sv[i]
                    r = t * 16 + i
                    for j in range(4):
                        outb_v[r, pl.ds(j * 16, 16)] = (
                            gpre_v[r, pl.ds(po + j * 16, 16)])
                        outb_v[r, pl.ds(EMB + j * 16, 16)] = (
                            gsuf_v[r, pl.ds(so + j * 16, 16)])

            pl.loop(0, chunk // 16)(row16)
            pltpu.sync_copy(outb_v, out_hbm.at[pl.ds(base + cb, chunk)])

        pl.loop(0, nchunks)(do_chunk)

    return k(pre, suf, pre2, suf2)


def kernel(unused, pre, suf, W_pre, W_suf):
    return _presuf_embed(pre, suf, W_pre, W_suf)


# speedup vs baseline: 2.9058x; 2.8013x over previous
import functools

import jax
import jax.numpy as jnp
from jax import lax
from jax.experimental import pallas as pl
from jax.experimental.pallas import tpu as pltpu
from jax.experimental.pallas import tpu_sc as plsc

EMB = 64
BATCH = 16384
LANES = 128
DEPTH = 4


@jax.jit
def _presuf_embed(pre, suf, W_pre, W_suf):
    info = plsc.get_sparse_core_info()
    nw = info.num_cores * info.num_subcores
    bpw = BATCH // nw
    ngroups = bpw // 16

    WTp = W_pre.T
    WTs = W_suf.T

    mesh = plsc.VectorSubcoreMesh(core_axis_name="c", subcore_axis_name="s")

    tile_t = pltpu.VMEM((EMB, LANES), jnp.float32)
    scratch = (
        [pltpu.VMEM((bpw,), jnp.int32), pltpu.VMEM((bpw,), jnp.int32)]
        + [tile_t for _ in range(2 * DEPTH)]
        + [pltpu.VMEM((64, 2 * EMB), jnp.float32)]
        + [pltpu.SemaphoreType.DMA for _ in range(2 * DEPTH)]
    )

    @functools.partial(
        pl.kernel,
        mesh=mesh,
        compiler_params=pltpu.CompilerParams(
            disable_bounds_checks=True, needs_layout_passes=False),
        out_type=jax.ShapeDtypeStruct((BATCH, 2 * EMB), jnp.float32),
        scratch_types=scratch,
    )
    def k(pre_hbm, suf_hbm, wtp_hbm, wts_hbm, out_hbm, *scr):
        pidx_v, sidx_v = scr[0], scr[1]
        rings = scr[2:2 + 2 * DEPTH]
        outb_v = scr[2 + 2 * DEPTH]
        sems = scr[3 + 2 * DEPTH:]

        wid = lax.axis_index("s") * info.num_cores + lax.axis_index("c")
        base = wid * bpw
        pltpu.sync_copy(pre_hbm.at[pl.ds(base, bpw)], pidx_v)
        pltpu.sync_copy(suf_hbm.at[pl.ds(base, bpw)], sidx_v)

        def issue(slot, p, s):
            tp = pl.multiple_of((p >> 7) * LANES, LANES)
            ts = pl.multiple_of((s >> 7) * LANES, LANES)
            pltpu.async_copy(
                wtp_hbm.at[:, pl.ds(tp, LANES)], rings[2 * slot],
                sems[2 * slot])
            pltpu.async_copy(
                wts_hbm.at[:, pl.ds(ts, LANES)], rings[2 * slot + 1],
                sems[2 * slot + 1])

        def wait(slot):
            pltpu.make_async_copy(
                wtp_hbm.at[:, pl.ds(0, LANES)], rings[2 * slot],
                sems[2 * slot]).wait()
            pltpu.make_async_copy(
                wts_hbm.at[:, pl.ds(0, LANES)], rings[2 * slot + 1],
                sems[2 * slot + 1]).wait()

        iot = lax.iota(jnp.int32, 16)

        def extract(slot, row64, p, s):
            lp = jnp.full((16,), p & (LANES - 1), jnp.int32)
            ls = jnp.full((16,), s & (LANES - 1), jnp.int32)
            for kq in range(4):
                fv = iot + (16 * kq)
                vp = plsc.load_gather(rings[2 * slot], [fv, lp])
                outb_v[row64, pl.ds(16 * kq, 16)] = vp
                vs = plsc.load_gather(rings[2 * slot + 1], [fv, ls])
                outb_v[row64, pl.ds(EMB + 16 * kq, 16)] = vs

        pv0 = pidx_v[pl.ds(0, 16)]
        sv0 = sidx_v[pl.ds(0, 16)]
        for il in range(DEPTH):
            issue(il % DEPTH, pv0[il], sv0[il])

        def group(g):
            gb = g * 16
            pv = pidx_v[pl.ds(gb, 16)]
            sv = sidx_v[pl.ds(gb, 16)]
            pvn = pidx_v[pl.ds(jnp.minimum(gb + 16, bpw - 16), 16)]
            svn = sidx_v[pl.ds(jnp.minimum(gb + 16, bpw - 16), 16)]
            for il in range(16):
                slot = il % DEPTH
                wait(slot)
                extract(slot, (g % 4) * 16 + il, pv[il], sv[il])
                nl = il + DEPTH
                if nl < 16:
                    issue(slot, pv[nl], sv[nl])
                else:
                    @pl.when(g < ngroups - 1)
                    def _():
                        issue(slot, pvn[nl - 16], svn[nl - 16])
            @pl.when((g % 4) == 3)
            def _():
                pltpu.sync_copy(
                    outb_v, out_hbm.at[pl.ds(base + (g - 3) * 16, 64)])

        pl.loop(0, ngroups)(group)

    return k(pre, suf, WTp, WTs)


def kernel(unused, pre, suf, W_pre, W_suf):
    return _presuf_embed(pre, suf, W_pre, W_suf)
